# baseline (speedup 1.0000x reference)
import jax
import jax.numpy as jnp
from jax import lax
from jax.experimental import pallas as pl
from jax.experimental.pallas import tpu as pltpu
from jax.experimental.pallas import tpu_sc as plsc

N_NODE = 10000
N_ACC = 10240
E = 320000
D = 128
B_OUT = 4096
NC = 2
NSUB = 16
C = 125
IDX_ROWS = E // C
ROWS_PER_TILE = IDX_ROWS // NSUB
CG = 160
CHUNKS = 128
E_PAD = NSUB * CHUNKS * CG
PAD_DST = N_ACC - 8
RBC = 16
NBATCH = CHUNKS // RBC
N_OUT_ROWS = N_ACC // NSUB

_SC_MESH = dict(core_axis_name="c", subcore_axis_name="s",
                num_cores=NC, num_subcores=NSUB)
_SC_PARAMS = pltpu.CompilerParams(needs_layout_passes=False)


def _make_aggr():
    mesh = plsc.VectorSubcoreMesh(**_SC_MESH)
    out_type = [jax.ShapeDtypeStruct((N_ACC, D), jnp.float32),
                jax.ShapeDtypeStruct((N_ACC, D), jnp.float32)]
    scratch = [pltpu.VMEM_SHARED((N_ACC, D), jnp.float32),
               pltpu.VMEM((RBC * CG,), jnp.int32),
               pltpu.VMEM((RBC * CG,), jnp.int32),
               pltpu.VMEM((CG, D), jnp.float32),
               pltpu.VMEM((CG, D), jnp.float32),
               pltpu.SemaphoreType.DMA,
               pltpu.SemaphoreType.DMA]

    def body(xu, xs, sb, db, sr, dr, out_s, out_u,
             acc, src_v, dst_v, rows0, rows1, gsem0, gsem1):
        core = lax.axis_index("c")
        sub = lax.axis_index("s")
        zero16 = jnp.zeros((16,), jnp.float32)

        def _z(i, carry):
            rows0[i // 8, pl.ds((i % 8) * 16, 16)] = zero16
            return carry
        lax.fori_loop(0, CG * 8, _z, 0)
        for q in range(N_OUT_ROWS // 160):
            pltpu.sync_copy(rows0.at[pl.ds(0, 160)],
                            acc.at[pl.ds(sub * N_OUT_ROWS + q * 160, 160)])
        plsc.subcore_barrier()

        def run(x_tab, src2d, dst2d, out):
            def gissue(c, rows, gsem):
                return pltpu.async_copy(
                    x_tab.at[src_v.at[pl.ds(c * CG, CG)]], rows, gsem)

            def gwait(rows, gsem):
                pltpu.make_async_copy(
                    x_tab.at[src_v.at[pl.ds(0, CG)]], rows, gsem).wait()

            def scat(c, rows):
                pltpu.sync_copy(rows, acc.at[dst_v.at[pl.ds(c * CG, CG)]],
                                add=True)

            for b in range(NBATCH):
                e0 = b * RBC * CG
                pltpu.sync_copy(src2d.at[sub, pl.ds(e0, RBC * CG)], src_v)
                pltpu.sync_copy(dst2d.at[sub, pl.ds(e0, RBC * CG)], dst_v)
                gissue(0, rows0, gsem0)

                def inner(i, carry):
                    c0 = 2 * i
                    gwait(rows0, gsem0)
                    d1 = gissue(c0 + 1, rows1, gsem1)
                    scat(c0, rows0)
                    d1.wait()
                    gissue(c0 + 2, rows0, gsem0)
                    scat(c0 + 1, rows1)
                    return carry
                lax.fori_loop(0, RBC // 2 - 1, inner, 0)
                gwait(rows0, gsem0)
                d1 = gissue(RBC - 1, rows1, gsem1)
                scat(RBC - 2, rows0)
                d1.wait()
                scat(RBC - 1, rows1)
            plsc.subcore_barrier()
            sl = pl.ds(sub * N_OUT_ROWS, N_OUT_ROWS)
            pltpu.sync_copy(acc.at[sl], out.at[sl])

        @pl.when(core == 0)
        def _():
            run(xu, sb, db, out_s)

        @pl.when(core == 1)
        def _():
            run(xs, sr, dr, out_u)

    return pl.kernel(body, out_type=out_type, mesh=mesh,
                     scratch_types=scratch, compiler_params=_SC_PARAMS)


_aggr = _make_aggr()


def _make_counts():
    mesh = plsc.VectorSubcoreMesh(**_SC_MESH)
    out_type = [jax.ShapeDtypeStruct((N_ACC,), jnp.float32),
                jax.ShapeDtypeStruct((N_ACC,), jnp.float32)]
    scratch = [pltpu.VMEM_SHARED((NSUB, N_ACC), jnp.float32),
               pltpu.VMEM((ROWS_PER_TILE, C), jnp.int32),
               pltpu.VMEM((N_ACC,), jnp.float32),
               pltpu.VMEM((N_OUT_ROWS,), jnp.float32),
               pltpu.VMEM((N_OUT_ROWS,), jnp.float32)]

    def body(db, dr, cnt_s_out, cnt_u_out, cnt_sh, dst_v, cnt_v, tbuf, cacc):
        core = lax.axis_index("c")
        sub = lax.axis_index("s")
        zero16 = jnp.zeros((16,), jnp.float32)
        one16 = jnp.full((16,), 1.0, jnp.float32)
        lane = lax.iota(jnp.int32, 16)

        def _zc(i, carry):
            cnt_v[pl.ds(i * 16, 16)] = zero16
            return carry
        lax.fori_loop(0, N_ACC // 16, _zc, 0)

        def run(dst2d, cnt_out):
            pltpu.sync_copy(
                dst2d.at[pl.ds(sub * ROWS_PER_TILE, ROWS_PER_TILE)], dst_v)

            def count_row(j, carry):
                for k in range(C // 16):
                    idx16 = dst_v[j, pl.ds(k * 16, 16)]
                    plsc.addupdate_scatter(cnt_v, [idx16], one16)
                m = lane >= (16 - (C - (C // 16) * 16))
                idx16 = dst_v[j, pl.ds(C - 16, 16)]
                idx16 = jnp.where(m, idx16, 0)
                plsc.addupdate_scatter(cnt_v, [idx16], one16, mask=m)
                return carry
            lax.fori_loop(0, ROWS_PER_TILE, count_row, 0)
            pltpu.sync_copy(cnt_v, cnt_sh.at[sub])
            plsc.subcore_barrier()
            sl = pl.ds(sub * N_OUT_ROWS, N_OUT_ROWS)

            def _czero(i, carry):
                cacc[pl.ds(i * 16, 16)] = zero16
                return carry
            lax.fori_loop(0, N_OUT_ROWS // 16, _czero, 0)
            for t in range(NSUB):
                pltpu.sync_copy(
                    cnt_sh.at[t, pl.ds(sub * N_OUT_ROWS, N_OUT_ROWS)], tbuf)

                def _cadd(i, carry):
                    s = pl.ds(i * 16, 16)
                    cacc[s] = cacc[s] + tbuf[s]
                    return carry
                lax.fori_loop(0, N_OUT_ROWS // 16, _cadd, 0)
            pltpu.sync_copy(cacc, cnt_out.at[sl])

        @pl.when(core == 0)
        def _():
            run(db, cnt_s_out)

        @pl.when(core == 1)
        def _():
            run(dr, cnt_u_out)

    return pl.kernel(body, out_type=out_type, mesh=mesh,
                     scratch_types=scratch, compiler_params=_SC_PARAMS)


_counts = _make_counts()


BLK = 1000


def _layer_body(sum_s, cnt_s, xs, bWl, bbl, bWr,
                sum_u, cnt_u, xu, rWl, rbl, rWr, out_s, out_u):
    cs = jnp.maximum(cnt_s[...], 1.0)
    hs = (jnp.dot(sum_s[...] / cs, bWl[...],
                  preferred_element_type=jnp.float32)
          + jnp.dot(xs[...], bWr[...], preferred_element_type=jnp.float32)
          + bbl[...])
    out_s[...] = jnp.maximum(hs, 0.0)
    cu = jnp.maximum(cnt_u[...], 1.0)
    hu = (jnp.dot(sum_u[...] / cu, rWl[...],
                  preferred_element_type=jnp.float32)
          + jnp.dot(xu[...], rWr[...], preferred_element_type=jnp.float32)
          + rbl[...])
    out_u[...] = jnp.maximum(hu, 0.0)


def _row_spec():
    return pl.BlockSpec((BLK, D), lambda i: (i, 0))


def _cnt_spec():
    return pl.BlockSpec((BLK, 1), lambda i: (i, 0))


def _w_spec():
    return pl.BlockSpec((D, D), lambda i: (0, 0))


def _b_spec():
    return pl.BlockSpec((1, D), lambda i: (0, 0))


_layer_tc = pl.pallas_call(
    _layer_body,
    grid=(N_NODE // BLK,),
    in_specs=[_row_spec(), _cnt_spec(), _row_spec(), _w_spec(), _b_spec(),
              _w_spec(),
              _row_spec(), _cnt_spec(), _row_spec(), _w_spec(), _b_spec(),
              _w_spec()],
    out_specs=[_row_spec(), _row_spec()],
    out_shape=[jax.ShapeDtypeStruct((N_NODE, D), jnp.float32),
               jax.ShapeDtypeStruct((N_NODE, D), jnp.float32)],
)


def _final_body(sum_s, cnt_s, xs, bWl, bbl, bWr, ws,
                sum_u, cnt_u, xu, rWl, rbl, rWr, wu, p_s, p_u):
    cs = jnp.maximum(cnt_s[...], 1.0)
    hs = (jnp.dot(sum_s[...] / cs, bWl[...],
                  preferred_element_type=jnp.float32)
          + jnp.dot(xs[...], bWr[...], preferred_element_type=jnp.float32)
          + bbl[...])
    p_s[...] = jnp.dot(jnp.maximum(hs, 0.0), ws[...],
                       preferred_element_type=jnp.float32)
    cu = jnp.maximum(cnt_u[...], 1.0)
    hu = (jnp.dot(sum_u[...] / cu, rWl[...],
                  preferred_element_type=jnp.float32)
          + jnp.dot(xu[...], rWr[...], preferred_element_type=jnp.float32)
          + rbl[...])
    p_u[...] = jnp.dot(jnp.maximum(hu, 0.0), wu[...],
                       preferred_element_type=jnp.float32)


_final_tc = pl.pallas_call(
    _final_body,
    grid=(N_NODE // BLK,),
    in_specs=[_row_spec(), _cnt_spec(), _row_spec(), _w_spec(), _b_spec(),
              _w_spec(), pl.BlockSpec((D, 1), lambda i: (0, 0)),
              _row_spec(), _cnt_spec(), _row_spec(), _w_spec(), _b_spec(),
              _w_spec(), pl.BlockSpec((D, 1), lambda i: (0, 0))],
    out_specs=[pl.BlockSpec((BLK, 1), lambda i: (i, 0)),
               pl.BlockSpec((BLK, 1), lambda i: (i, 0))],
    out_shape=[jax.ShapeDtypeStruct((N_NODE, 1), jnp.float32),
               jax.ShapeDtypeStruct((N_NODE, 1), jnp.float32)],
)


PER_TILE = B_OUT // (NC * NSUB)


def _make_final_sc():
    mesh = plsc.VectorSubcoreMesh(**_SC_MESH)
    scratch = [pltpu.VMEM((N_NODE,), jnp.float32),
               pltpu.VMEM((N_NODE,), jnp.float32),
               pltpu.VMEM((PER_TILE,), jnp.int32),
               pltpu.VMEM((PER_TILE,), jnp.int32),
               pltpu.VMEM((16,), jnp.float32),
               pltpu.VMEM((PER_TILE,), jnp.float32)]

    def body(pu_hbm, ps_hbm, m0_hbm, m1_hbm, b_hbm, out_hbm,
             pu_v, ps_v, m0_v, m1_v, b_v, out_v):
        core = lax.axis_index("c")
        sub = lax.axis_index("s")
        wid = sub * NC + core
        pltpu.sync_copy(pu_hbm, pu_v)
        pltpu.sync_copy(ps_hbm, ps_v)
        pltpu.sync_copy(m0_hbm.at[pl.ds(wid * PER_TILE, PER_TILE)], m0_v)
        pltpu.sync_copy(m1_hbm.at[pl.ds(wid * PER_TILE, PER_TILE)], m1_v)
        pltpu.sync_copy(b_hbm, b_v)
        b = b_v[pl.ds(0, 16)]
        for k in range(PER_TILE // 16):
            i0 = m0_v[pl.ds(k * 16, 16)]
            i1 = m1_v[pl.ds(k * 16, 16)]
            a = (plsc.load_gather(pu_v, [i0])
                 + plsc.load_gather(ps_v, [i1]) + b)
            out_v[pl.ds(k * 16, 16)] = 1.0 / (1.0 + jnp.exp(-a))
        pltpu.sync_copy(out_v, out_hbm.at[pl.ds(wid * PER_TILE, PER_TILE)])

    return pl.kernel(
        body, out_type=jax.ShapeDtypeStruct((B_OUT,), jnp.float32),
        mesh=mesh, scratch_types=scratch, compiler_params=_SC_PARAMS)


_final_sc = _make_final_sc()


def kernel(x_user, x_seller, edge_index_buy, edge_index_rev, mask,
           l0_buy_Wl, l0_buy_bl, l0_buy_Wr, l0_rev_Wl, l0_rev_bl, l0_rev_Wr,
           l1_buy_Wl, l1_buy_bl, l1_buy_Wr, l1_rev_Wl, l1_rev_bl, l1_rev_Wr,
           lin_W, lin_b):
    sb = edge_index_buy[0].astype(jnp.int32)
    db = edge_index_buy[1].astype(jnp.int32)
    sr = edge_index_rev[0].astype(jnp.int32)
    dr = edge_index_rev[1].astype(jnp.int32)
    m0 = mask[:, 0].astype(jnp.int32)
    m1 = mask[:, 1].astype(jnp.int32)

    def pad2(v, fill):
        p = jnp.full((E_PAD - E,), fill, jnp.int32)
        return jnp.concatenate([v, p]).reshape(NSUB, CHUNKS * CG)

    sb2, db2 = pad2(sb, 0), pad2(db, PAD_DST)
    sr2, dr2 = pad2(sr, 0), pad2(dr, PAD_DST)

    cnt_s, cnt_u = _counts(db.reshape(IDX_ROWS, C), dr.reshape(IDX_ROWS, C))
    cnt_s = cnt_s.reshape(N_ACC, 1)
    cnt_u = cnt_u.reshape(N_ACC, 1)
    sum_s, sum_u = _aggr(x_user, x_seller, sb2, db2, sr2, dr2)
    xs1, xu1 = _layer_tc(
        sum_s, cnt_s, x_seller, l0_buy_Wl, l0_buy_bl.reshape(1, D), l0_buy_Wr,
        sum_u, cnt_u, x_user, l0_rev_Wl, l0_rev_bl.reshape(1, D), l0_rev_Wr)
    sum_s1, sum_u1 = _aggr(xu1, xs1, sb2, db2, sr2, dr2)
    ps, pu = _final_tc(
        sum_s1, cnt_s, xs1, l1_buy_Wl, l1_buy_bl.reshape(1, D), l1_buy_Wr,
        lin_W[D:],
        sum_u1, cnt_u, xu1, l1_rev_Wl, l1_rev_bl.reshape(1, D), l1_rev_Wr,
        lin_W[:D])
    b16 = jnp.broadcast_to(lin_b.reshape(1), (16,)).astype(jnp.float32)
    return _final_sc(pu.reshape(-1), ps.reshape(-1), m0, m1, b16)


# speedup vs baseline: 2.0931x; 2.0931x over previous
import jax
import jax.numpy as jnp
from jax import lax
from jax.experimental import pallas as pl
from jax.experimental.pallas import tpu as pltpu
from jax.experimental.pallas import tpu_sc as plsc

N_NODE = 10000
N_ACC = 10240
E = 320000
D = 128
B_OUT = 4096
NC = 2
NSUB = 16
C = 125
IDX_ROWS = E // C
ROWS_PER_TILE = IDX_ROWS // NSUB
RB = 40
NBATCH = ROWS_PER_TILE // RB
N_OUT_ROWS = N_ACC // NSUB

_SC_MESH = dict(core_axis_name="c", subcore_axis_name="s",
                num_cores=NC, num_subcores=NSUB)
_SC_PARAMS = pltpu.CompilerParams(needs_layout_passes=False)


def _make_aggr():
    mesh = plsc.VectorSubcoreMesh(**_SC_MESH)
    out_type = [jax.ShapeDtypeStruct((N_ACC, D), jnp.float32),
                jax.ShapeDtypeStruct((N_ACC, D), jnp.float32)]
    scratch = [pltpu.VMEM_SHARED((N_ACC, D), jnp.float32),
               pltpu.VMEM((RB, C), jnp.int32),
               pltpu.VMEM((RB, C), jnp.int32),
               pltpu.VMEM((C, D), jnp.float32),
               pltpu.VMEM((C, D), jnp.float32),
               pltpu.SemaphoreType.DMA,
               pltpu.SemaphoreType.DMA]

    def body(xu, xs, sb, db, sr, dr, out_s, out_u,
             acc, src_v, dst_v, rows0, rows1, gsem0, gsem1):
        core = lax.axis_index("c")
        sub = lax.axis_index("s")
        zero16 = jnp.zeros((16,), jnp.float32)

        def _z(i, carry):
            rows0[i // 8, pl.ds((i % 8) * 16, 16)] = zero16
            return carry
        lax.fori_loop(0, C * 8, _z, 0)
        for q in range(8):
            pltpu.sync_copy(rows0.at[pl.ds(0, 80)],
                            acc.at[pl.ds(sub * N_OUT_ROWS + q * 80, 80)])
        plsc.subcore_barrier()

        def run(x_tab, src2d, dst2d, out):
            def gissue(c, rows, gsem):
                return pltpu.async_copy(x_tab.at[src_v.at[c]], rows, gsem)

            def gwait(rows, gsem):
                pltpu.make_async_copy(x_tab.at[src_v.at[0]], rows,
                                      gsem).wait()

            def scat(c, rows):
                pltpu.sync_copy(rows, acc.at[dst_v.at[c]], add=True)

            for b in range(NBATCH):
                row0 = sub * ROWS_PER_TILE + b * RB
                pltpu.sync_copy(src2d.at[pl.ds(row0, RB)], src_v)
                pltpu.sync_copy(dst2d.at[pl.ds(row0, RB)], dst_v)
                gissue(0, rows0, gsem0)

                def inner(i, carry):
                    c0 = 2 * i
                    gwait(rows0, gsem0)
                    d1 = gissue(c0 + 1, rows1, gsem1)
                    scat(c0, rows0)
                    d1.wait()
                    gissue(c0 + 2, rows0, gsem0)
                    scat(c0 + 1, rows1)
                    return carry
                lax.fori_loop(0, RB // 2 - 1, inner, 0)
                gwait(rows0, gsem0)
                d1 = gissue(RB - 1, rows1, gsem1)
                scat(RB - 2, rows0)
                d1.wait()
                scat(RB - 1, rows1)
            plsc.subcore_barrier()
            sl = pl.ds(sub * N_OUT_ROWS, N_OUT_ROWS)
            pltpu.sync_copy(acc.at[sl], out.at[sl])

        @pl.when(core == 0)
        def _():
            run(xu, sb, db, out_s)

        @pl.when(core == 1)
        def _():
            run(xs, sr, dr, out_u)

    return pl.kernel(body, out_type=out_type, mesh=mesh,
                     scratch_types=scratch, compiler_params=_SC_PARAMS)


_aggr = _make_aggr()


def _make_counts():
    mesh = plsc.VectorSubcoreMesh(**_SC_MESH)
    out_type = [jax.ShapeDtypeStruct((N_ACC,), jnp.float32),
                jax.ShapeDtypeStruct((N_ACC,), jnp.float32)]
    scratch = [pltpu.VMEM_SHARED((NSUB, N_ACC), jnp.float32),
               pltpu.VMEM((ROWS_PER_TILE, C), jnp.int32),
               pltpu.VMEM((N_ACC,), jnp.float32),
               pltpu.VMEM((N_OUT_ROWS,), jnp.float32),
               pltpu.VMEM((N_OUT_ROWS,), jnp.float32)]

    def body(db, dr, cnt_s_out, cnt_u_out, cnt_sh, dst_v, cnt_v, tbuf, cacc):
        core = lax.axis_index("c")
        sub = lax.axis_index("s")
        zero16 = jnp.zeros((16,), jnp.float32)
        one16 = jnp.full((16,), 1.0, jnp.float32)
        lane = lax.iota(jnp.int32, 16)

        def _zc(i, carry):
            cnt_v[pl.ds(i * 16, 16)] = zero16
            return carry
        lax.fori_loop(0, N_ACC // 16, _zc, 0)

        def run(dst2d, cnt_out):
            pltpu.sync_copy(
                dst2d.at[pl.ds(sub * ROWS_PER_TILE, ROWS_PER_TILE)], dst_v)

            def count_row(j, carry):
                for k in range(C // 16):
                    idx16 = dst_v[j, pl.ds(k * 16, 16)]
                    plsc.addupdate_scatter(cnt_v, [idx16], one16)
                m = lane >= (16 - (C - (C // 16) * 16))
                idx16 = dst_v[j, pl.ds(C - 16, 16)]
                idx16 = jnp.where(m, idx16, 0)
                plsc.addupdate_scatter(cnt_v, [idx16], one16, mask=m)
                return carry
            lax.fori_loop(0, ROWS_PER_TILE, count_row, 0)
            pltpu.sync_copy(cnt_v, cnt_sh.at[sub])
            plsc.subcore_barrier()
            sl = pl.ds(sub * N_OUT_ROWS, N_OUT_ROWS)

            def _czero(i, carry):
                cacc[pl.ds(i * 16, 16)] = zero16
                return carry
            lax.fori_loop(0, N_OUT_ROWS // 16, _czero, 0)
            for t in range(NSUB):
                pltpu.sync_copy(
                    cnt_sh.at[t, pl.ds(sub * N_OUT_ROWS, N_OUT_ROWS)], tbuf)

                def _cadd(i, carry):
                    s = pl.ds(i * 16, 16)
                    cacc[s] = cacc[s] + tbuf[s]
                    return carry
                lax.fori_loop(0, N_OUT_ROWS // 16, _cadd, 0)
            pltpu.sync_copy(cacc, cnt_out.at[sl])

        @pl.when(core == 0)
        def _():
            run(db, cnt_s_out)

        @pl.when(core == 1)
        def _():
            run(dr, cnt_u_out)

    return pl.kernel(body, out_type=out_type, mesh=mesh,
                     scratch_types=scratch, compiler_params=_SC_PARAMS)


_counts = _make_counts()


BLK = 1000


def _layer_body(sum_s, cnt_s, xs, bWl, bbl, bWr,
                sum_u, cnt_u, xu, rWl, rbl, rWr, out_s, out_u):
    cs = jnp.maximum(cnt_s[...], 1.0)
    hs = (jnp.dot(sum_s[...] / cs, bWl[...],
                  preferred_element_type=jnp.float32)
          + jnp.dot(xs[...], bWr[...], preferred_element_type=jnp.float32)
          + bbl[...])
    out_s[...] = jnp.maximum(hs, 0.0)
    cu = jnp.maximum(cnt_u[...], 1.0)
    hu = (jnp.dot(sum_u[...] / cu, rWl[...],
                  preferred_element_type=jnp.float32)
          + jnp.dot(xu[...], rWr[...], preferred_element_type=jnp.float32)
          + rbl[...])
    out_u[...] = jnp.maximum(hu, 0.0)


def _row_spec():
    return pl.BlockSpec((BLK, D), lambda i: (i, 0))


def _cnt_spec():
    return pl.BlockSpec((BLK, 1), lambda i: (i, 0))


def _w_spec():
    return pl.BlockSpec((D, D), lambda i: (0, 0))


def _b_spec():
    return pl.BlockSpec((1, D), lambda i: (0, 0))


_layer_tc = pl.pallas_call(
    _layer_body,
    grid=(N_NODE // BLK,),
    in_specs=[_row_spec(), _cnt_spec(), _row_spec(), _w_spec(), _b_spec(),
              _w_spec(),
              _row_spec(), _cnt_spec(), _row_spec(), _w_spec(), _b_spec(),
              _w_spec()],
    out_specs=[_row_spec(), _row_spec()],
    out_shape=[jax.ShapeDtypeStruct((N_NODE, D), jnp.float32),
               jax.ShapeDtypeStruct((N_NODE, D), jnp.float32)],
)


def _final_body(sum_s, cnt_s, xs, bWl, bbl, bWr, ws,
                sum_u, cnt_u, xu, rWl, rbl, rWr, wu, p_s, p_u):
    cs = jnp.maximum(cnt_s[...], 1.0)
    hs = (jnp.dot(sum_s[...] / cs, bWl[...],
                  preferred_element_type=jnp.float32)
          + jnp.dot(xs[...], bWr[...], preferred_element_type=jnp.float32)
          + bbl[...])
    p_s[...] = jnp.dot(jnp.maximum(hs, 0.0), ws[...],
                       preferred_element_type=jnp.float32)
    cu = jnp.maximum(cnt_u[...], 1.0)
    hu = (jnp.dot(sum_u[...] / cu, rWl[...],
                  preferred_element_type=jnp.float32)
          + jnp.dot(xu[...], rWr[...], preferred_element_type=jnp.float32)
          + rbl[...])
    p_u[...] = jnp.dot(jnp.maximum(hu, 0.0), wu[...],
                       preferred_element_type=jnp.float32)


_final_tc = pl.pallas_call(
    _final_body,
    grid=(N_NODE // BLK,),
    in_specs=[_row_spec(), _cnt_spec(), _row_spec(), _w_spec(), _b_spec(),
              _w_spec(), pl.BlockSpec((D, 1), lambda i: (0, 0)),
              _row_spec(), _cnt_spec(), _row_spec(), _w_spec(), _b_spec(),
              _w_spec(), pl.BlockSpec((D, 1), lambda i: (0, 0))],
    out_specs=[pl.BlockSpec((BLK, 1), lambda i: (i, 0)),
               pl.BlockSpec((BLK, 1), lambda i: (i, 0))],
    out_shape=[jax.ShapeDtypeStruct((N_NODE, 1), jnp.float32),
               jax.ShapeDtypeStruct((N_NODE, 1), jnp.float32)],
)


PER_TILE = B_OUT // (NC * NSUB)


def _make_final_sc():
    mesh = plsc.VectorSubcoreMesh(**_SC_MESH)
    scratch = [pltpu.VMEM((N_NODE,), jnp.float32),
               pltpu.VMEM((N_NODE,), jnp.float32),
               pltpu.VMEM((PER_TILE,), jnp.int32),
               pltpu.VMEM((PER_TILE,), jnp.int32),
               pltpu.VMEM((16,), jnp.float32),
               pltpu.VMEM((PER_TILE,), jnp.float32)]

    def body(pu_hbm, ps_hbm, m0_hbm, m1_hbm, b_hbm, out_hbm,
             pu_v, ps_v, m0_v, m1_v, b_v, out_v):
        core = lax.axis_index("c")
        sub = lax.axis_index("s")
        wid = sub * NC + core
        pltpu.sync_copy(pu_hbm, pu_v)
        pltpu.sync_copy(ps_hbm, ps_v)
        pltpu.sync_copy(m0_hbm.at[pl.ds(wid * PER_TILE, PER_TILE)], m0_v)
        pltpu.sync_copy(m1_hbm.at[pl.ds(wid * PER_TILE, PER_TILE)], m1_v)
        pltpu.sync_copy(b_hbm, b_v)
        b = b_v[pl.ds(0, 16)]
        for k in range(PER_TILE // 16):
            i0 = m0_v[pl.ds(k * 16, 16)]
            i1 = m1_v[pl.ds(k * 16, 16)]
            a = (plsc.load_gather(pu_v, [i0])
                 + plsc.load_gather(ps_v, [i1]) + b)
            out_v[pl.ds(k * 16, 16)] = 1.0 / (1.0 + jnp.exp(-a))
        pltpu.sync_copy(out_v, out_hbm.at[pl.ds(wid * PER_TILE, PER_TILE)])

    return pl.kernel(
        body, out_type=jax.ShapeDtypeStruct((B_OUT,), jnp.float32),
        mesh=mesh, scratch_types=scratch, compiler_params=_SC_PARAMS)


_final_sc = _make_final_sc()


def kernel(x_user, x_seller, edge_index_buy, edge_index_rev, mask,
           l0_buy_Wl, l0_buy_bl, l0_buy_Wr, l0_rev_Wl, l0_rev_bl, l0_rev_Wr,
           l1_buy_Wl, l1_buy_bl, l1_buy_Wr, l1_rev_Wl, l1_rev_bl, l1_rev_Wr,
           lin_W, lin_b):
    sb = edge_index_buy[0].astype(jnp.int32).reshape(IDX_ROWS, C)
    db = edge_index_buy[1].astype(jnp.int32).reshape(IDX_ROWS, C)
    sr = edge_index_rev[0].astype(jnp.int32).reshape(IDX_ROWS, C)
    dr = edge_index_rev[1].astype(jnp.int32).reshape(IDX_ROWS, C)
    m0 = mask[:, 0].astype(jnp.int32)
    m1 = mask[:, 1].astype(jnp.int32)

    cnt_s, cnt_u = _counts(db, dr)
    cnt_s = cnt_s.reshape(N_ACC, 1)
    cnt_u = cnt_u.reshape(N_ACC, 1)
    sum_s, sum_u = _aggr(x_user, x_seller, sb, db, sr, dr)
    xs1, xu1 = _layer_tc(
        sum_s, cnt_s, x_seller, l0_buy_Wl, l0_buy_bl.reshape(1, D), l0_buy_Wr,
        sum_u, cnt_u, x_user, l0_rev_Wl, l0_rev_bl.reshape(1, D), l0_rev_Wr)
    sum_s1, sum_u1 = _aggr(xu1, xs1, sb, db, sr, dr)
    ps, pu = _final_tc(
        sum_s1, cnt_s, xs1, l1_buy_Wl, l1_buy_bl.reshape(1, D), l1_buy_Wr,
        lin_W[D:],
        sum_u1, cnt_u, xu1, l1_rev_Wl, l1_rev_bl.reshape(1, D), l1_rev_Wr,
        lin_W[:D])
    b16 = jnp.broadcast_to(lin_b.reshape(1), (16,)).astype(jnp.float32)
    return _final_sc(pu.reshape(-1), ps.reshape(-1), m0, m1, b16)


# speedup vs baseline: 2.4788x; 1.1843x over previous
import jax
import jax.numpy as jnp
from jax import lax
from jax.experimental import pallas as pl
from jax.experimental.pallas import tpu as pltpu
from jax.experimental.pallas import tpu_sc as plsc

N_NODE = 10000
N_ACC = 10240
E = 320000
D = 128
B_OUT = 4096
NC = 2
NSUB = 16
C = 125
IDX_ROWS = E // C
ROWS_PER_TILE = IDX_ROWS // NSUB
RB = 40
NBATCH = ROWS_PER_TILE // RB
N_OUT_ROWS = N_ACC // NSUB

_SC_MESH = dict(core_axis_name="c", subcore_axis_name="s",
                num_cores=NC, num_subcores=NSUB)
_SC_PARAMS = pltpu.CompilerParams(needs_layout_passes=False)


def _make_aggr():
    mesh = plsc.VectorSubcoreMesh(**_SC_MESH)
    out_type = [jax.ShapeDtypeStruct((N_ACC, D), jnp.float32),
                jax.ShapeDtypeStruct((N_ACC, D), jnp.float32)]
    scratch = [pltpu.VMEM_SHARED((N_ACC, D), jnp.float32),
               pltpu.VMEM((RB, C), jnp.int32),
               pltpu.VMEM((RB, C), jnp.int32),
               pltpu.VMEM((C, D), jnp.float32),
               pltpu.VMEM((C, D), jnp.float32),
               pltpu.SemaphoreType.DMA]

    def body(xu, xs, sb, db, sr, dr, out_s, out_u,
             acc, src_v, dst_v, rows0, rows1, gsem):
        core = lax.axis_index("c")
        sub = lax.axis_index("s")
        zero16 = jnp.zeros((16,), jnp.float32)

        def _z(i, carry):
            rows0[i // 8, pl.ds((i % 8) * 16, 16)] = zero16
            return carry
        lax.fori_loop(0, C * 8, _z, 0)
        for q in range(8):
            pltpu.sync_copy(rows0.at[pl.ds(0, 80)],
                            acc.at[pl.ds(sub * N_OUT_ROWS + q * 80, 80)])
        plsc.subcore_barrier()

        def run(x_tab, src2d, dst2d, out):
            def gissue(c, rows):
                return pltpu.async_copy(x_tab.at[src_v.at[c]], rows, gsem)

            def gwait(rows):
                pltpu.make_async_copy(x_tab.at[src_v.at[0]], rows,
                                      gsem).wait()

            def scat(c, rows):
                pltpu.sync_copy(rows, acc.at[dst_v.at[c]], add=True)

            for b in range(NBATCH):
                row0 = sub * ROWS_PER_TILE + b * RB
                pltpu.sync_copy(src2d.at[pl.ds(row0, RB)], src_v)
                pltpu.sync_copy(dst2d.at[pl.ds(row0, RB)], dst_v)
                gissue(0, rows0)
                gissue(1, rows1)

                def inner(i, carry):
                    c0 = 2 * i
                    gwait(rows0)
                    scat(c0, rows0)
                    gissue(c0 + 2, rows0)
                    gwait(rows1)
                    scat(c0 + 1, rows1)
                    gissue(c0 + 3, rows1)
                    return carry
                lax.fori_loop(0, RB // 2 - 1, inner, 0)
                gwait(rows0)
                scat(RB - 2, rows0)
                gwait(rows1)
                scat(RB - 1, rows1)
            plsc.subcore_barrier()
            sl = pl.ds(sub * N_OUT_ROWS, N_OUT_ROWS)
            pltpu.sync_copy(acc.at[sl], out.at[sl])

        @pl.when(core == 0)
        def _():
            run(xu, sb, db, out_s)

        @pl.when(core == 1)
        def _():
            run(xs, sr, dr, out_u)

    return pl.kernel(body, out_type=out_type, mesh=mesh,
                     scratch_types=scratch, compiler_params=_SC_PARAMS)


_aggr = _make_aggr()


def _make_counts():
    mesh = plsc.VectorSubcoreMesh(**_SC_MESH)
    out_type = [jax.ShapeDtypeStruct((N_ACC,), jnp.float32),
                jax.ShapeDtypeStruct((N_ACC,), jnp.float32)]
    scratch = [pltpu.VMEM_SHARED((NSUB, N_ACC), jnp.float32),
               pltpu.VMEM((ROWS_PER_TILE, C), jnp.int32),
               pltpu.VMEM((N_ACC,), jnp.float32),
               pltpu.VMEM((N_OUT_ROWS,), jnp.float32),
               pltpu.VMEM((N_OUT_ROWS,), jnp.float32)]

    def body(db, dr, cnt_s_out, cnt_u_out, cnt_sh, dst_v, cnt_v, tbuf, cacc):
        core = lax.axis_index("c")
        sub = lax.axis_index("s")
        zero16 = jnp.zeros((16,), jnp.float32)
        one16 = jnp.full((16,), 1.0, jnp.float32)
        lane = lax.iota(jnp.int32, 16)

        def _zc(i, carry):
            cnt_v[pl.ds(i * 16, 16)] = zero16
            return carry
        lax.fori_loop(0, N_ACC // 16, _zc, 0)

        def run(dst2d, cnt_out):
            pltpu.sync_copy(
                dst2d.at[pl.ds(sub * ROWS_PER_TILE, ROWS_PER_TILE)], dst_v)

            def count_row(j, carry):
                for k in range(C // 16):
                    idx16 = dst_v[j, pl.ds(k * 16, 16)]
                    plsc.addupdate_scatter(cnt_v, [idx16], one16)
                m = lane >= (16 - (C - (C // 16) * 16))
                idx16 = dst_v[j, pl.ds(C - 16, 16)]
                idx16 = jnp.where(m, idx16, 0)
                plsc.addupdate_scatter(cnt_v, [idx16], one16, mask=m)
                return carry
            lax.fori_loop(0, ROWS_PER_TILE, count_row, 0)
            pltpu.sync_copy(cnt_v, cnt_sh.at[sub])
            plsc.subcore_barrier()
            sl = pl.ds(sub * N_OUT_ROWS, N_OUT_ROWS)

            def _czero(i, carry):
                cacc[pl.ds(i * 16, 16)] = zero16
                return carry
            lax.fori_loop(0, N_OUT_ROWS // 16, _czero, 0)
            for t in range(NSUB):
                pltpu.sync_copy(
                    cnt_sh.at[t, pl.ds(sub * N_OUT_ROWS, N_OUT_ROWS)], tbuf)

                def _cadd(i, carry):
                    s = pl.ds(i * 16, 16)
                    cacc[s] = cacc[s] + tbuf[s]
                    return carry
                lax.fori_loop(0, N_OUT_ROWS // 16, _cadd, 0)
            pltpu.sync_copy(cacc, cnt_out.at[sl])

        @pl.when(core == 0)
        def _():
            run(db, cnt_s_out)

        @pl.when(core == 1)
        def _():
            run(dr, cnt_u_out)

    return pl.kernel(body, out_type=out_type, mesh=mesh,
                     scratch_types=scratch, compiler_params=_SC_PARAMS)


_counts = _make_counts()


BLK = 1000


def _layer_body(sum_s, cnt_s, xs, bWl, bbl, bWr,
                sum_u, cnt_u, xu, rWl, rbl, rWr, out_s, out_u):
    cs = jnp.maximum(cnt_s[...], 1.0)
    hs = (jnp.dot(sum_s[...] / cs, bWl[...],
                  preferred_element_type=jnp.float32)
          + jnp.dot(xs[...], bWr[...], preferred_element_type=jnp.float32)
          + bbl[...])
    out_s[...] = jnp.maximum(hs, 0.0)
    cu = jnp.maximum(cnt_u[...], 1.0)
    hu = (jnp.dot(sum_u[...] / cu, rWl[...],
                  preferred_element_type=jnp.float32)
          + jnp.dot(xu[...], rWr[...], preferred_element_type=jnp.float32)
          + rbl[...])
    out_u[...] = jnp.maximum(hu, 0.0)


def _row_spec():
    return pl.BlockSpec((BLK, D), lambda i: (i, 0))


def _cnt_spec():
    return pl.BlockSpec((BLK, 1), lambda i: (i, 0))


def _w_spec():
    return pl.BlockSpec((D, D), lambda i: (0, 0))


def _b_spec():
    return pl.BlockSpec((1, D), lambda i: (0, 0))


_layer_tc = pl.pallas_call(
    _layer_body,
    grid=(N_NODE // BLK,),
    in_specs=[_row_spec(), _cnt_spec(), _row_spec(), _w_spec(), _b_spec(),
              _w_spec(),
              _row_spec(), _cnt_spec(), _row_spec(), _w_spec(), _b_spec(),
              _w_spec()],
    out_specs=[_row_spec(), _row_spec()],
    out_shape=[jax.ShapeDtypeStruct((N_NODE, D), jnp.float32),
               jax.ShapeDtypeStruct((N_NODE, D), jnp.float32)],
)


def _final_body(sum_s, cnt_s, xs, bWl, bbl, bWr, ws,
                sum_u, cnt_u, xu, rWl, rbl, rWr, wu, p_s, p_u):
    cs = jnp.maximum(cnt_s[...], 1.0)
    hs = (jnp.dot(sum_s[...] / cs, bWl[...],
                  preferred_element_type=jnp.float32)
          + jnp.dot(xs[...], bWr[...], preferred_element_type=jnp.float32)
          + bbl[...])
    p_s[...] = jnp.dot(jnp.maximum(hs, 0.0), ws[...],
                       preferred_element_type=jnp.float32)
    cu = jnp.maximum(cnt_u[...], 1.0)
    hu = (jnp.dot(sum_u[...] / cu, rWl[...],
                  preferred_element_type=jnp.float32)
          + jnp.dot(xu[...], rWr[...], preferred_element_type=jnp.float32)
          + rbl[...])
    p_u[...] = jnp.dot(jnp.maximum(hu, 0.0), wu[...],
                       preferred_element_type=jnp.float32)


_final_tc = pl.pallas_call(
    _final_body,
    grid=(N_NODE // BLK,),
    in_specs=[_row_spec(), _cnt_spec(), _row_spec(), _w_spec(), _b_spec(),
              _w_spec(), pl.BlockSpec((D, 1), lambda i: (0, 0)),
              _row_spec(), _cnt_spec(), _row_spec(), _w_spec(), _b_spec(),
              _w_spec(), pl.BlockSpec((D, 1), lambda i: (0, 0))],
    out_specs=[pl.BlockSpec((BLK, 1), lambda i: (i, 0)),
               pl.BlockSpec((BLK, 1), lambda i: (i, 0))],
    out_shape=[jax.ShapeDtypeStruct((N_NODE, 1), jnp.float32),
               jax.ShapeDtypeStruct((N_NODE, 1), jnp.float32)],
)


PER_TILE = B_OUT // (NC * NSUB)


def _make_final_sc():
    mesh = plsc.VectorSubcoreMesh(**_SC_MESH)
    scratch = [pltpu.VMEM((N_NODE,), jnp.float32),
               pltpu.VMEM((N_NODE,), jnp.float32),
               pltpu.VMEM((PER_TILE,), jnp.int32),
               pltpu.VMEM((PER_TILE,), jnp.int32),
               pltpu.VMEM((16,), jnp.float32),
               pltpu.VMEM((PER_TILE,), jnp.float32)]

    def body(pu_hbm, ps_hbm, m0_hbm, m1_hbm, b_hbm, out_hbm,
             pu_v, ps_v, m0_v, m1_v, b_v, out_v):
        core = lax.axis_index("c")
        sub = lax.axis_index("s")
        wid = sub * NC + core
        pltpu.sync_copy(pu_hbm, pu_v)
        pltpu.sync_copy(ps_hbm, ps_v)
        pltpu.sync_copy(m0_hbm.at[pl.ds(wid * PER_TILE, PER_TILE)], m0_v)
        pltpu.sync_copy(m1_hbm.at[pl.ds(wid * PER_TILE, PER_TILE)], m1_v)
        pltpu.sync_copy(b_hbm, b_v)
        b = b_v[pl.ds(0, 16)]
        for k in range(PER_TILE // 16):
            i0 = m0_v[pl.ds(k * 16, 16)]
            i1 = m1_v[pl.ds(k * 16, 16)]
            a = (plsc.load_gather(pu_v, [i0])
                 + plsc.load_gather(ps_v, [i1]) + b)
            out_v[pl.ds(k * 16, 16)] = 1.0 / (1.0 + jnp.exp(-a))
        pltpu.sync_copy(out_v, out_hbm.at[pl.ds(wid * PER_TILE, PER_TILE)])

    return pl.kernel(
        body, out_type=jax.ShapeDtypeStruct((B_OUT,), jnp.float32),
        mesh=mesh, scratch_types=scratch, compiler_params=_SC_PARAMS)


_final_sc = _make_final_sc()


def kernel(x_user, x_seller, edge_index_buy, edge_index_rev, mask,
           l0_buy_Wl, l0_buy_bl, l0_buy_Wr, l0_rev_Wl, l0_rev_bl, l0_rev_Wr,
           l1_buy_Wl, l1_buy_bl, l1_buy_Wr, l1_rev_Wl, l1_rev_bl, l1_rev_Wr,
           lin_W, lin_b):
    sb = edge_index_buy[0].astype(jnp.int32).reshape(IDX_ROWS, C)
    db = edge_index_buy[1].astype(jnp.int32).reshape(IDX_ROWS, C)
    sr = edge_index_rev[0].astype(jnp.int32).reshape(IDX_ROWS, C)
    dr = edge_index_rev[1].astype(jnp.int32).reshape(IDX_ROWS, C)
    m0 = mask[:, 0].astype(jnp.int32)
    m1 = mask[:, 1].astype(jnp.int32)

    cnt_s, cnt_u = _counts(db, dr)
    cnt_s = cnt_s.reshape(N_ACC, 1)
    cnt_u = cnt_u.reshape(N_ACC, 1)
    sum_s, sum_u = _aggr(x_user, x_seller, sb, db, sr, dr)
    xs1, xu1 = _layer_tc(
        sum_s, cnt_s, x_seller, l0_buy_Wl, l0_buy_bl.reshape(1, D), l0_buy_Wr,
        sum_u, cnt_u, x_user, l0_rev_Wl, l0_rev_bl.reshape(1, D), l0_rev_Wr)
    sum_s1, sum_u1 = _aggr(xu1, xs1, sb, db, sr, dr)
    ps, pu = _final_tc(
        sum_s1, cnt_s, xs1, l1_buy_Wl, l1_buy_bl.reshape(1, D), l1_buy_Wr,
        lin_W[D:],
        sum_u1, cnt_u, xu1, l1_rev_Wl, l1_rev_bl.reshape(1, D), l1_rev_Wr,
        lin_W[:D])
    b16 = jnp.broadcast_to(lin_b.reshape(1), (16,)).astype(jnp.float32)
    return _final_sc(pu.reshape(-1), ps.reshape(-1), m0, m1, b16)
